# Initial kernel scaffold; baseline (speedup 1.0000x reference)
import jax
import jax.numpy as jnp
from jax.experimental import pallas as pl


def kernel(hidden_states, router_w, w_gate, w_up, w_down, ws_gate, ws_up, ws_down, w_shared_gate):
    raise NotImplementedError("write your pallas kernel here")


# speedup vs baseline: 1.2236x; 1.2236x over previous
import functools

import jax
import jax.numpy as jnp
from jax.experimental import pallas as pl
from jax.experimental.pallas import tpu as pltpu


def _router_body(top_k, x_ref, w_ref, rw_ref):
    x = x_ref[...]
    logits = jax.lax.dot_general(
        x, w_ref[...], (((1,), (1,)), ((), ())),
        preferred_element_type=jnp.float32)
    m = jnp.max(logits, axis=1, keepdims=True)
    ex = jnp.exp(logits - m)
    probs = ex / jnp.sum(ex, axis=1, keepdims=True)

    e = probs.shape[1]
    lane = jax.lax.broadcasted_iota(jnp.int32, probs.shape, 1)
    sel = jnp.zeros(probs.shape, dtype=jnp.bool_)
    for _ in range(top_k):
        cur = jnp.where(sel, -jnp.inf, probs)
        mx = jnp.max(cur, axis=1, keepdims=True)
        ismax = jnp.logical_and(cur == mx, jnp.logical_not(sel))
        first = jnp.min(jnp.where(ismax, lane, e), axis=1, keepdims=True)
        sel = jnp.logical_or(sel, lane == first)
    kept = jnp.where(sel, probs, 0.0)
    rw_ref[...] = kept / jnp.sum(kept, axis=1, keepdims=True)


def _routing_weights(x, router_w, top_k):
    t, d = x.shape
    e = router_w.shape[0]
    bt = min(512, t)
    return pl.pallas_call(
        functools.partial(_router_body, top_k),
        grid=(t // bt,),
        in_specs=[
            pl.BlockSpec((bt, d), lambda i: (i, 0)),
            pl.BlockSpec((e, d), lambda i: (0, 0)),
        ],
        out_specs=pl.BlockSpec((bt, e), lambda i: (i, 0)),
        out_shape=jax.ShapeDtypeStruct((t, e), jnp.float32),
    )(x, router_w)


def _expert_body(x_ref, wg_ref, wu_ref, wd_ref, rw_ref, out_ref):
    e = pl.program_id(1)
    x = x_ref[...]
    g = jax.lax.dot_general(x, wg_ref[0], (((1,), (1,)), ((), ())),
                            preferred_element_type=jnp.float32)
    u = jax.lax.dot_general(x, wu_ref[0], (((1,), (1,)), ((), ())),
                            preferred_element_type=jnp.float32)
    h = (g * jax.nn.sigmoid(g) * u).astype(jnp.bfloat16)
    y = jax.lax.dot_general(h, wd_ref[0], (((1,), (1,)), ((), ())),
                            preferred_element_type=jnp.float32)
    rw = rw_ref[...]
    lane = jax.lax.broadcasted_iota(jnp.int32, rw.shape, 1)
    scale = jnp.sum(jnp.where(lane == e, rw, 0.0), axis=1, keepdims=True)

    @pl.when(e == 0)
    def _():
        out_ref[...] = scale * y

    @pl.when(e != 0)
    def _():
        out_ref[...] += scale * y


def _expert_out(x16, w_gate, w_up, w_down, rw):
    t, d = x16.shape
    e, f, _ = w_gate.shape
    bt = min(512, t)
    return pl.pallas_call(
        _expert_body,
        grid=(t // bt, e),
        in_specs=[
            pl.BlockSpec((bt, d), lambda i, j: (i, 0)),
            pl.BlockSpec((1, f, d), lambda i, j: (j, 0, 0)),
            pl.BlockSpec((1, f, d), lambda i, j: (j, 0, 0)),
            pl.BlockSpec((1, d, f), lambda i, j: (j, 0, 0)),
            pl.BlockSpec((bt, e), lambda i, j: (i, 0)),
        ],
        out_specs=pl.BlockSpec((bt, d), lambda i, j: (i, 0)),
        out_shape=jax.ShapeDtypeStruct((t, d), jnp.float32),
    )(x16, w_gate, w_up, w_down, rw)


def _shared_body(nf, x_ref, wsg_ref, wsu_ref, wsd_ref, wshg_ref, eo_ref,
                 out_ref, acc_ref):
    j = pl.program_id(1)
    x = x_ref[...]
    g = jax.lax.dot_general(x, wsg_ref[...], (((1,), (1,)), ((), ())),
                            preferred_element_type=jnp.float32)
    u = jax.lax.dot_general(x, wsu_ref[...], (((1,), (1,)), ((), ())),
                            preferred_element_type=jnp.float32)
    h = (g * jax.nn.sigmoid(g) * u).astype(jnp.bfloat16)
    part = jax.lax.dot_general(h, wsd_ref[...], (((1,), (1,)), ((), ())),
                               preferred_element_type=jnp.float32)

    @pl.when(j == 0)
    def _():
        acc_ref[...] = part

    @pl.when(j != 0)
    def _():
        acc_ref[...] += part

    @pl.when(j == nf - 1)
    def _():
        gl = jnp.sum(x.astype(jnp.float32) * wshg_ref[...].astype(jnp.float32),
                     axis=1, keepdims=True)
        out_ref[...] = eo_ref[...] + jax.nn.sigmoid(gl) * acc_ref[...]


def _shared_combine(x16, ws_gate, ws_up, ws_down, w_shared_gate, expert_out):
    t, d = x16.shape
    f_sh = ws_gate.shape[0]
    bt = min(256, t)
    bf = 1408 if f_sh % 1408 == 0 else f_sh
    nf = f_sh // bf
    return pl.pallas_call(
        functools.partial(_shared_body, nf),
        grid=(t // bt, nf),
        in_specs=[
            pl.BlockSpec((bt, d), lambda i, j: (i, 0)),
            pl.BlockSpec((bf, d), lambda i, j: (j, 0)),
            pl.BlockSpec((bf, d), lambda i, j: (j, 0)),
            pl.BlockSpec((d, bf), lambda i, j: (0, j)),
            pl.BlockSpec((1, d), lambda i, j: (0, 0)),
            pl.BlockSpec((bt, d), lambda i, j: (i, 0)),
        ],
        out_specs=pl.BlockSpec((bt, d), lambda i, j: (i, 0)),
        out_shape=jax.ShapeDtypeStruct((t, d), jnp.float32),
        scratch_shapes=[pltpu.VMEM((bt, d), jnp.float32)],
    )(x16, ws_gate, ws_up, ws_down, w_shared_gate, expert_out)


def kernel(hidden_states, router_w, w_gate, w_up, w_down, ws_gate, ws_up,
           ws_down, w_shared_gate):
    b, s, d = hidden_states.shape
    x = hidden_states.reshape(-1, d)
    top_k = 8

    rw = _routing_weights(x, router_w, top_k)

    x16 = x.astype(jnp.bfloat16)
    eo = _expert_out(x16, w_gate.astype(jnp.bfloat16),
                     w_up.astype(jnp.bfloat16), w_down.astype(jnp.bfloat16),
                     rw)
    out = _shared_combine(x16, ws_gate.astype(jnp.bfloat16),
                          ws_up.astype(jnp.bfloat16),
                          ws_down.astype(jnp.bfloat16),
                          w_shared_gate.astype(jnp.bfloat16), eo)
    return out.reshape(b, s, d)
